# Initial kernel scaffold; baseline (speedup 1.0000x reference)
import jax
import jax.numpy as jnp
from jax.experimental import pallas as pl


def kernel(f, knn_idx, W1, b1, W2, b2, W3, b3, W4, b4, W5, b5):
    raise NotImplementedError("write your pallas kernel here")


# speedup vs baseline: 46.4236x; 46.4236x over previous
import dataclasses

import jax
import jax.numpy as jnp
from jax import lax
from jax.experimental import pallas as pl
from jax.experimental.pallas import tpu as pltpu
from jax.experimental.pallas import tpu_sc as plsc

_TP = 1024
_NC = 2
_NS = 16


def _sc_gather(f3flat, idxflat):
    B = f3flat.shape[0]
    n3 = f3flat.shape[1]
    ek = idxflat.shape[1]
    nw = _NC * _NS
    ipw = ek // nw
    opw = ipw * 3
    mesh = plsc.VectorSubcoreMesh(core_axis_name="c", subcore_axis_name="s")
    cp = pltpu.CompilerParams()
    if "needs_layout_passes" in pltpu.CompilerParams.__dataclass_fields__:
        cp = dataclasses.replace(cp, needs_layout_passes=False)

    @pl.kernel(
        out_type=jax.ShapeDtypeStruct((B, ek * 3), jnp.float32),
        mesh=mesh,
        compiler_params=cp,
        scratch_types=[
            pltpu.VMEM((n3,), jnp.float32),
            pltpu.VMEM((ipw,), jnp.int32),
            pltpu.VMEM((opw,), jnp.float32),
        ],
    )
    def gather_kernel(f_hbm, i_hbm, o_hbm, tab_v, idx_v, out_v):
        wid = lax.axis_index("s") * _NC + lax.axis_index("c")
        lanes = lax.iota(jnp.int32, 16)
        for b in range(B):
            pltpu.sync_copy(f_hbm.at[b], tab_v)
            pltpu.sync_copy(i_hbm.at[b, pl.ds(wid * ipw, ipw)], idx_v)

            @pl.loop(0, ipw // 16)
            def _(p):
                knn = idx_v[pl.ds(p * 16, 16)]
                addr = knn * 3
                g0 = plsc.load_gather(tab_v, [addr])
                g1 = plsc.load_gather(tab_v, [addr + 1])
                g2 = plsc.load_gather(tab_v, [addr + 2])
                si = lanes * 3 + p * 48
                plsc.store_scatter(out_v, [si], g0)
                plsc.store_scatter(out_v, [si + 1], g1)
                plsc.store_scatter(out_v, [si + 2], g2)

            pltpu.sync_copy(out_v, o_hbm.at[b, pl.ds(wid * opw, opw)])

    return gather_kernel(f3flat, idxflat)


def _dense_body(g_ref, f3_ref, wcat_ref, b1_ref, w2b_ref, b2_ref,
                w3_ref, b3_ref, w4_ref, b4_ref, w5_ref, out_ref):
    g = g_ref[0]
    f3 = f3_ref[0]
    xin = jnp.concatenate([g, f3], axis=1)
    h = jnp.dot(xin, wcat_ref[...], preferred_element_type=jnp.float32)
    h = jnp.maximum(h + b1_ref[...], 0.0)
    h = jnp.dot(h, w2b_ref[...], preferred_element_type=jnp.float32)
    h = jnp.maximum(h + b2_ref[...], 0.0)
    m = jnp.maximum(h[:, :256], h[:, 256:])
    m = jnp.maximum(m[:, :128], m[:, 128:])
    m = jnp.maximum(m[:, :64], m[:, 64:])
    xmax = jnp.maximum(m[:, :32], m[:, 32:])
    t = jnp.dot(f3, w3_ref[...], preferred_element_type=jnp.float32)
    t = jnp.maximum(t + b3_ref[...], 0.0)
    gsk = jnp.dot(t, w4_ref[...], preferred_element_type=jnp.float32)
    gsk = jnp.maximum(gsk + b4_ref[...], 0.0)
    s = xmax + gsk
    out_ref[0] = jnp.dot(s, w5_ref[...], preferred_element_type=jnp.float32)


def _blkdiag(w, r):
    a, b = w.shape
    out = jnp.zeros((r * a, r * b), w.dtype)
    for i in range(r):
        out = out.at[i * a:(i + 1) * a, i * b:(i + 1) * b].set(w)
    return out


def kernel(f, knn_idx, W1, b1, W2, b2, W3, b3, W4, b4, W5, b5):
    B, N, C = f.shape
    K = knn_idx.shape[2]

    g3 = _sc_gather(f.reshape(B, N * C), knn_idx.reshape(B, N * K))
    g3r = g3.reshape(B, N, K * C)

    W1a = W1[:C] + W1[C:]
    W1b = W1[C:]
    wcat = jnp.concatenate([_blkdiag(W1a, K), jnp.tile(-W1b, (1, K))], axis=0)
    b1r = jnp.tile(b1, K)[None]
    w2b = _blkdiag(W2, K)
    b2r = jnp.tile(b2, K)[None]

    grid = (B, N // _TP)
    out = pl.pallas_call(
        _dense_body,
        grid=grid,
        in_specs=[
            pl.BlockSpec((1, _TP, K * C), lambda b, i: (b, i, 0)),
            pl.BlockSpec((1, _TP, 3), lambda b, i: (b, i, 0)),
            pl.BlockSpec((K * C + 3, K * 32), lambda b, i: (0, 0)),
            pl.BlockSpec((1, K * 32), lambda b, i: (0, 0)),
            pl.BlockSpec((K * 32, K * 32), lambda b, i: (0, 0)),
            pl.BlockSpec((1, K * 32), lambda b, i: (0, 0)),
            pl.BlockSpec((3, 32), lambda b, i: (0, 0)),
            pl.BlockSpec((1, 32), lambda b, i: (0, 0)),
            pl.BlockSpec((32, 32), lambda b, i: (0, 0)),
            pl.BlockSpec((1, 32), lambda b, i: (0, 0)),
            pl.BlockSpec((32, 3), lambda b, i: (0, 0)),
        ],
        out_specs=pl.BlockSpec((1, _TP, 3), lambda b, i: (b, i, 0)),
        out_shape=jax.ShapeDtypeStruct((B, N, 3), jnp.float32),
    )(g3r, f, wcat, b1r, w2b, b2r, W3, b3[None], W4, b4[None], W5)
    return out + b5


# speedup vs baseline: 55.1876x; 1.1888x over previous
import dataclasses

import jax
import jax.numpy as jnp
from jax import lax
from jax.experimental import pallas as pl
from jax.experimental.pallas import tpu as pltpu
from jax.experimental.pallas import tpu_sc as plsc

_TP = 1024
_NC = 2
_NS = 16


def _sc_gather(f, knn_idx):
    B, N, C = f.shape
    K = knn_idx.shape[2]
    nw = _NC * _NS
    ppw = N // nw
    mesh = plsc.VectorSubcoreMesh(core_axis_name="c", subcore_axis_name="s")
    cp = pltpu.CompilerParams()
    if "needs_layout_passes" in pltpu.CompilerParams.__dataclass_fields__:
        cp = dataclasses.replace(cp, needs_layout_passes=False)

    @pl.kernel(
        out_type=jax.ShapeDtypeStruct((B, N, K * C), jnp.float32),
        mesh=mesh,
        compiler_params=cp,
        scratch_types=[
            pltpu.VMEM((N * C,), jnp.float32),
            pltpu.VMEM((N * C,), jnp.float32),
            pltpu.VMEM((ppw * K,), jnp.int32),
            pltpu.VMEM((ppw * K,), jnp.int32),
            pltpu.VMEM((ppw, K * C), jnp.float32),
            pltpu.VMEM((ppw, K * C), jnp.float32),
            pltpu.SemaphoreType.DMA((2,)),
            pltpu.SemaphoreType.DMA((2,)),
            pltpu.SemaphoreType.DMA((2,)),
        ],
    )
    def gather_kernel(f_hbm, i_hbm, o_hbm, tab0, tab1, idx0, idx1,
                      out0, out1, tsem, isem, osem):
        tab_v = [tab0, tab1]
        idx_v = [idx0, idx1]
        out_v = [out0, out1]
        wid = lax.axis_index("s") * _NC + lax.axis_index("c")
        base = wid * ppw
        lanes3 = lax.iota(jnp.int32, 16) * C

        def start_in(b, s):
            t = pltpu.async_copy(f_hbm.at[b], tab_v[s], tsem.at[s])
            i = pltpu.async_copy(i_hbm.at[b, pl.ds(base * K, ppw * K)],
                                 idx_v[s], isem.at[s])
            return t, i

        in_cp = {0: start_in(0, 0)}
        out_cp = {}
        for b in range(B):
            s = b % 2
            if b + 1 < B:
                in_cp[b + 1] = start_in(b + 1, 1 - s)
            t, i = in_cp.pop(b)
            t.wait()
            i.wait()
            if b >= 2:
                out_cp.pop(b - 2).wait()

            @pl.loop(0, ppw)
            def _(p):
                knn = idx_v[s][pl.ds(p * K, K)]
                addr = knn * C
                g0 = plsc.load_gather(tab_v[s], [addr])
                g1 = plsc.load_gather(tab_v[s], [addr + 1])
                g2 = plsc.load_gather(tab_v[s], [addr + 2])
                row = out_v[s].at[p]
                plsc.store_scatter(row, [lanes3], g0)
                plsc.store_scatter(row, [lanes3 + 1], g1)
                plsc.store_scatter(row, [lanes3 + 2], g2)

            out_cp[b] = pltpu.async_copy(
                out_v[s], o_hbm.at[b, pl.ds(base, ppw)], osem.at[s])
        out_cp.pop(B - 2).wait()
        out_cp.pop(B - 1).wait()

    idx2 = knn_idx.reshape(B, N * K)
    return gather_kernel(f.reshape(B, N * C), idx2)


def _dense_body(g_ref, f3_ref, wcat_ref, b1_ref, w2b_ref, b2_ref,
                w3_ref, b3_ref, w4_ref, b4_ref, w5_ref, b5_ref, out_ref):
    g = g_ref[0]
    f3 = f3_ref[0]
    xin = jnp.concatenate([g, f3], axis=1)
    h = jnp.dot(xin, wcat_ref[...], preferred_element_type=jnp.float32)
    h = jnp.maximum(h + b1_ref[...], 0.0)
    h = jnp.dot(h, w2b_ref[...], preferred_element_type=jnp.float32)
    h = jnp.maximum(h + b2_ref[...], 0.0)
    m = jnp.maximum(h[:, :256], h[:, 256:])
    m = jnp.maximum(m[:, :128], m[:, 128:])
    m = jnp.maximum(m[:, :64], m[:, 64:])
    xmax = jnp.maximum(m[:, :32], m[:, 32:])
    t = jnp.dot(f3, w3_ref[...], preferred_element_type=jnp.float32)
    t = jnp.maximum(t + b3_ref[...], 0.0)
    gsk = jnp.dot(t, w4_ref[...], preferred_element_type=jnp.float32)
    gsk = jnp.maximum(gsk + b4_ref[...], 0.0)
    s = xmax + gsk
    out_ref[0] = (jnp.dot(s, w5_ref[...], preferred_element_type=jnp.float32)
                  + b5_ref[...])


def _blkdiag(w, r):
    a, b = w.shape
    out = jnp.zeros((r * a, r * b), w.dtype)
    for i in range(r):
        out = out.at[i * a:(i + 1) * a, i * b:(i + 1) * b].set(w)
    return out


def kernel(f, knn_idx, W1, b1, W2, b2, W3, b3, W4, b4, W5, b5):
    B, N, C = f.shape
    K = knn_idx.shape[2]

    g3r = _sc_gather(f, knn_idx)

    W1a = W1[:C] + W1[C:]
    W1b = W1[C:]
    wcat = jnp.concatenate([_blkdiag(W1a, K), jnp.tile(-W1b, (1, K))], axis=0)
    b1r = jnp.tile(b1, K)[None]
    w2b = _blkdiag(W2, K)
    b2r = jnp.tile(b2, K)[None]

    grid = (B, N // _TP)
    out = pl.pallas_call(
        _dense_body,
        grid=grid,
        in_specs=[
            pl.BlockSpec((1, _TP, K * C), lambda b, i: (b, i, 0)),
            pl.BlockSpec((1, _TP, 3), lambda b, i: (b, i, 0)),
            pl.BlockSpec((K * C + 3, K * 32), lambda b, i: (0, 0)),
            pl.BlockSpec((1, K * 32), lambda b, i: (0, 0)),
            pl.BlockSpec((K * 32, K * 32), lambda b, i: (0, 0)),
            pl.BlockSpec((1, K * 32), lambda b, i: (0, 0)),
            pl.BlockSpec((3, 32), lambda b, i: (0, 0)),
            pl.BlockSpec((1, 32), lambda b, i: (0, 0)),
            pl.BlockSpec((32, 32), lambda b, i: (0, 0)),
            pl.BlockSpec((1, 32), lambda b, i: (0, 0)),
            pl.BlockSpec((32, 3), lambda b, i: (0, 0)),
            pl.BlockSpec((1, 3), lambda b, i: (0, 0)),
        ],
        out_specs=pl.BlockSpec((1, _TP, 3), lambda b, i: (b, i, 0)),
        out_shape=jax.ShapeDtypeStruct((B, N, 3), jnp.float32),
    )(g3r, f, wcat, b1r, w2b, b2r, W3, b3[None], W4, b4[None], W5, b5[None])
    return out


# speedup vs baseline: 62.9111x; 1.1400x over previous
import dataclasses

import jax
import jax.numpy as jnp
from jax import lax
from jax.experimental import pallas as pl
from jax.experimental.pallas import tpu as pltpu
from jax.experimental.pallas import tpu_sc as plsc

_TP = 1024
_NC = 2
_NS = 16


def _prep_body(f_ref, i_ref, ft_ref, it_ref):
    fr = f_ref[0]
    z = jnp.zeros((fr.shape[0], 5), jnp.float32)
    ft_ref[0] = jnp.concatenate([fr, z], axis=1).T[:3]
    it_ref[0] = i_ref[0].T


def _prep(f, knn_idx):
    B, N, C = f.shape
    K = knn_idx.shape[2]
    return pl.pallas_call(
        _prep_body,
        grid=(B,),
        in_specs=[
            pl.BlockSpec((1, N, C), lambda b: (b, 0, 0)),
            pl.BlockSpec((1, N, K), lambda b: (b, 0, 0)),
        ],
        out_specs=[
            pl.BlockSpec((1, C, N), lambda b: (b, 0, 0)),
            pl.BlockSpec((1, K, N), lambda b: (b, 0, 0)),
        ],
        out_shape=[
            jax.ShapeDtypeStruct((B, C, N), jnp.float32),
            jax.ShapeDtypeStruct((B, K, N), jnp.int32),
        ],
    )(f, knn_idx)


def _sc_gather(fT, idxT):
    B, C, N = fT.shape
    K = idxT.shape[1]
    nw = _NC * _NS
    ppw = N // nw
    mesh = plsc.VectorSubcoreMesh(core_axis_name="c", subcore_axis_name="s")
    cp = pltpu.CompilerParams()
    if "needs_layout_passes" in pltpu.CompilerParams.__dataclass_fields__:
        cp = dataclasses.replace(cp, needs_layout_passes=False)

    @pl.kernel(
        out_type=jax.ShapeDtypeStruct((B, N, K * C), jnp.float32),
        mesh=mesh,
        compiler_params=cp,
        scratch_types=[
            pltpu.VMEM((C, N), jnp.float32),
            pltpu.VMEM((C, N), jnp.float32),
            pltpu.VMEM((K, ppw), jnp.int32),
            pltpu.VMEM((K, ppw), jnp.int32),
            pltpu.VMEM((ppw, K * C), jnp.float32),
            pltpu.SemaphoreType.DMA((2,)),
            pltpu.SemaphoreType.DMA((2,)),
            pltpu.SemaphoreType.DMA((2,)),
        ],
    )
    def gather_kernel(f_hbm, i_hbm, o_hbm, tab0, tab1, idx0, idx1,
                      out0, tsem, isem, osem):
        tab_v = [tab0, tab1]
        idx_v = [idx0, idx1]
        wid = lax.axis_index("s") * _NC + lax.axis_index("c")
        base = wid * ppw
        lanes = lax.iota(jnp.int32, 16)
        lanes3 = lanes * C
        c0 = jnp.zeros((16,), jnp.int32)
        c1 = jnp.full((16,), 1, jnp.int32)
        c2 = jnp.full((16,), 2, jnp.int32)

        def start_in(b, s):
            t = pltpu.async_copy(f_hbm.at[b], tab_v[s], tsem.at[s])
            i = pltpu.async_copy(i_hbm.at[b, :, pl.ds(base, ppw)],
                                 idx_v[s], isem.at[s])
            return t, i

        in_cp = {0: start_in(0, 0)}
        out_cp = {}
        for b in range(B):
            s = b % 2
            if b + 1 < B:
                in_cp[b + 1] = start_in(b + 1, 1 - s)
            t, i = in_cp.pop(b)
            t.wait()
            i.wait()
            if b >= 1:
                out_cp.pop(b - 1).wait()

            @pl.loop(0, ppw)
            def _(p):
                pv = jax.lax.broadcast(p, (16,))
                knn = plsc.load_gather(idx_v[s], [lanes, pv])
                g0 = plsc.load_gather(tab_v[s], [c0, knn])
                g1 = plsc.load_gather(tab_v[s], [c1, knn])
                g2 = plsc.load_gather(tab_v[s], [c2, knn])
                row = out0.at[p]
                plsc.store_scatter(row, [lanes3], g0)
                plsc.store_scatter(row, [lanes3 + 1], g1)
                plsc.store_scatter(row, [lanes3 + 2], g2)

            out_cp[b] = pltpu.async_copy(
                out0, o_hbm.at[b, pl.ds(base, ppw)], osem.at[b % 2])
        out_cp.pop(B - 1).wait()

    return gather_kernel(fT, idxT)


def _dense_body(g_ref, f3_ref, wcat_ref, b1_ref, w2b_ref, b2_ref,
                w3_ref, b3_ref, w4_ref, b4_ref, w5_ref, b5_ref, out_ref):
    g = g_ref[0]
    f3 = f3_ref[0]
    xin = jnp.concatenate([g, f3], axis=1)
    h = jnp.dot(xin.astype(jnp.bfloat16), wcat_ref[...],
                preferred_element_type=jnp.float32)
    h = jnp.maximum(h + b1_ref[...], 0.0)
    h = jnp.dot(h.astype(jnp.bfloat16), w2b_ref[...],
                preferred_element_type=jnp.float32)
    h = jnp.maximum(h + b2_ref[...], 0.0)
    m = jnp.maximum(h[:, :256], h[:, 256:])
    m = jnp.maximum(m[:, :128], m[:, 128:])
    m = jnp.maximum(m[:, :64], m[:, 64:])
    xmax = jnp.maximum(m[:, :32], m[:, 32:])
    t = jnp.dot(f3, w3_ref[...], preferred_element_type=jnp.float32)
    t = jnp.maximum(t + b3_ref[...], 0.0)
    gsk = jnp.dot(t, w4_ref[...], preferred_element_type=jnp.float32)
    gsk = jnp.maximum(gsk + b4_ref[...], 0.0)
    s = xmax + gsk
    out_ref[0] = (jnp.dot(s, w5_ref[...], preferred_element_type=jnp.float32)
                  + b5_ref[...])


def _blkdiag(w, r):
    a, b = w.shape
    out = jnp.zeros((r * a, r * b), w.dtype)
    for i in range(r):
        out = out.at[i * a:(i + 1) * a, i * b:(i + 1) * b].set(w)
    return out


def kernel(f, knn_idx, W1, b1, W2, b2, W3, b3, W4, b4, W5, b5):
    B, N, C = f.shape
    K = knn_idx.shape[2]

    fT, idxT = _prep(f, knn_idx)
    g3r = _sc_gather(fT, idxT)

    W1a = W1[:C] + W1[C:]
    W1b = W1[C:]
    wcat = jnp.concatenate([_blkdiag(W1a, K),
                            jnp.tile(-W1b, (1, K))], axis=0).astype(jnp.bfloat16)
    b1r = jnp.tile(b1, K)[None]
    w2b = _blkdiag(W2, K).astype(jnp.bfloat16)
    b2r = jnp.tile(b2, K)[None]

    grid = (B, N // _TP)
    out = pl.pallas_call(
        _dense_body,
        grid=grid,
        in_specs=[
            pl.BlockSpec((1, _TP, K * C), lambda b, i: (b, i, 0)),
            pl.BlockSpec((1, _TP, 3), lambda b, i: (b, i, 0)),
            pl.BlockSpec((K * C + 3, K * 32), lambda b, i: (0, 0)),
            pl.BlockSpec((1, K * 32), lambda b, i: (0, 0)),
            pl.BlockSpec((K * 32, K * 32), lambda b, i: (0, 0)),
            pl.BlockSpec((1, K * 32), lambda b, i: (0, 0)),
            pl.BlockSpec((3, 32), lambda b, i: (0, 0)),
            pl.BlockSpec((1, 32), lambda b, i: (0, 0)),
            pl.BlockSpec((32, 32), lambda b, i: (0, 0)),
            pl.BlockSpec((1, 32), lambda b, i: (0, 0)),
            pl.BlockSpec((32, 3), lambda b, i: (0, 0)),
            pl.BlockSpec((1, 3), lambda b, i: (0, 0)),
        ],
        out_specs=pl.BlockSpec((1, _TP, 3), lambda b, i: (b, i, 0)),
        out_shape=jax.ShapeDtypeStruct((B, N, 3), jnp.float32),
    )(g3r, f, wcat, b1r, w2b, b2r, W3, b3[None], W4, b4[None], W5, b5[None])
    return out
